# baseline (speedup 1.0000x reference)
import functools

import jax
import jax.numpy as jnp
from jax import lax
from jax.experimental import pallas as pl
from jax.experimental.pallas import tpu as pltpu
from jax.experimental.pallas import tpu_sc as plsc

N = 256
K = 4
N_S = 128
N_A = 16
N_FC = 128
N_H = 128

B = 32
NB = N // B

NW = 32
BPW = (N * K) // NW
APW = N // NW
CH = 64


def _sc_gather(ob, fp, states, js_flat):
  mesh = plsc.VectorSubcoreMesh(core_axis_name="c", subcore_axis_name="s")

  @functools.partial(
      pl.kernel, mesh=mesh,
      out_type=[
          jax.ShapeDtypeStruct((N * K, N_S), jnp.float32),
          jax.ShapeDtypeStruct((N * K, 128), jnp.float32),
          jax.ShapeDtypeStruct((N * K, 2 * N_H), jnp.float32),
      ],
      scratch_types=[
          pltpu.VMEM((BPW,), jnp.int32),
          pltpu.VMEM((BPW, N_S), jnp.float32),
          pltpu.VMEM((BPW, 128), jnp.float32),
          pltpu.VMEM((BPW, 2 * N_H), jnp.float32),
          pltpu.SemaphoreType.DMA,
          pltpu.SemaphoreType.DMA,
          pltpu.SemaphoreType.DMA,
      ],
  )
  def gather_kernel(ob_hbm, fp_hbm, st_hbm, js_hbm, nx_hbm, pf_hbm, ms_hbm,
                    idx_v, a_v, b_v, c_v, s0, s1, s2):
    wid = lax.axis_index("s") * 2 + lax.axis_index("c")
    base = wid * BPW
    pltpu.sync_copy(js_hbm.at[pl.ds(base, BPW)], idx_v)
    cp0 = pltpu.async_copy(ob_hbm.at[idx_v], a_v, s0)
    cp1 = pltpu.async_copy(fp_hbm.at[idx_v], b_v, s1)
    cp2 = pltpu.async_copy(st_hbm.at[idx_v], c_v, s2)
    cp0.wait()
    o0 = pltpu.async_copy(a_v, nx_hbm.at[pl.ds(base, BPW)], s0)
    cp1.wait()
    o1 = pltpu.async_copy(b_v, pf_hbm.at[pl.ds(base, BPW)], s1)
    cp2.wait()
    o2 = pltpu.async_copy(c_v, ms_hbm.at[pl.ds(base, BPW)], s2)
    o0.wait()
    o1.wait()
    o2.wait()

  return gather_kernel(ob, fp, states, js_flat)


def _sc_pwh(pf_rows, Wp, states, Wh):
  mesh = plsc.VectorSubcoreMesh(core_axis_name="c", subcore_axis_name="s")

  @functools.partial(
      pl.kernel, mesh=mesh,
      out_type=[
          jax.ShapeDtypeStruct((N, N_FC), jnp.float32),
          jax.ShapeDtypeStruct((N, 4 * N_H), jnp.float32),
      ],
      scratch_types=[
          pltpu.VMEM((K * APW, 128), jnp.float32),
          pltpu.VMEM((K * N_A, N_FC), jnp.float32),
          pltpu.VMEM((K * N_A, N_FC), jnp.float32),
          pltpu.VMEM((APW, N_FC), jnp.float32),
          pltpu.VMEM((APW, 2 * N_H), jnp.float32),
          pltpu.VMEM((CH, 4 * N_H), jnp.float32),
          pltpu.VMEM((CH, 4 * N_H), jnp.float32),
          pltpu.VMEM((APW, 4 * N_H), jnp.float32),
          pltpu.SemaphoreType.DMA,
          pltpu.SemaphoreType.DMA,
          pltpu.SemaphoreType.DMA,
          pltpu.SemaphoreType.DMA,
      ],
  )
  def pwh_kernel(pf_hbm, wp_hbm, st_hbm, wh_hbm, sp_hbm, gh_hbm,
                 pf_v, p0, p1, op_v, h_all, w0, w1, out_v, e0, e1, s0, s1):
    wid = lax.axis_index("s") * 2 + lax.axis_index("c")
    a0 = wid * APW
    pltpu.sync_copy(pf_hbm.at[pl.ds(a0 * K, K * APW)], pf_v)
    pltpu.sync_copy(st_hbm.at[pl.ds(a0, APW), :], h_all)

    pbufs = (p0, p1)
    psems = (e0, e1)
    pltpu.async_copy(wp_hbm.at[a0], p0, e0)
    for a in range(APW):
      pltpu.make_async_copy(wp_hbm.at[a0 + a], pbufs[a % 2],
                            psems[a % 2]).wait()
      if a + 1 < APW:
        pltpu.async_copy(wp_hbm.at[a0 + a + 1], pbufs[(a + 1) % 2],
                         psems[(a + 1) % 2])
      w_r = pbufs[a % 2]
      acc = tuple(jnp.zeros((16,), jnp.float32) for _ in range(8))

      def pbody(j, acc, a=a, w_r=w_r):
        hv = pf_v[K * a + j, pl.ds(0, 16)]
        for u in range(16):
          hi = hv[u]
          acc = tuple(acc[o] + hi * w_r[j * 16 + u, pl.ds(o * 16, 16)]
                      for o in range(8))
        return acc

      acc = lax.fori_loop(0, K, pbody, acc)
      for o in range(8):
        op_v[a, pl.ds(o * 16, 16)] = acc[o]

    bufs = (w0, w1)
    sems = (s0, s1)

    def issue(a, c):
      return pltpu.async_copy(
          wh_hbm.at[a0 + a, pl.ds(c * CH, CH), :], bufs[c], sems[c])

    def compute_chunk(a, c, w_r):
      for og in range(4):
        if c == 0:
          acc = tuple(jnp.zeros((16,), jnp.float32) for _ in range(8))
        else:
          acc = tuple(out_v[a, pl.ds(og * 128 + o * 16, 16)]
                      for o in range(8))

        def body(j, acc, a=a, c=c, og=og, w_r=w_r):
          hv = h_all[a, pl.ds(c * CH + j * 16, 16)]
          for u in range(16):
            hi = hv[u]
            acc = tuple(
                acc[o] + hi * w_r[j * 16 + u, pl.ds(og * 128 + o * 16, 16)]
                for o in range(8))
          return acc

        acc = lax.fori_loop(0, CH // 16, body, acc)
        for o in range(8):
          out_v[a, pl.ds(og * 128 + o * 16, 16)] = acc[o]

    issue(0, 0)

    def agent_body(a, carry):
      pltpu.make_async_copy(
          wh_hbm.at[a0 + a, pl.ds(0, CH), :], w0, s0).wait()
      issue(a, 1)
      compute_chunk(a, 0, w0)
      pltpu.make_async_copy(
          wh_hbm.at[a0 + a, pl.ds(CH, CH), :], w1, s1).wait()

      @pl.when(a < APW - 1)
      def _():
        issue(a + 1, 0)

      compute_chunk(a, 1, w1)
      return carry

    lax.fori_loop(0, APW, agent_body, 0)
    oc0 = pltpu.async_copy(op_v, sp_hbm.at[pl.ds(a0, APW)], e0)
    oc1 = pltpu.async_copy(out_v, gh_hbm.at[pl.ds(a0, APW)], e1)
    oc0.wait()
    oc1.wait()

  return pwh_kernel(pf_rows, Wp, states, Wh)


def _dot(u, v):
  return jax.lax.dot_general(
      u, v, (((1,), (0,)), ((), ())),
      precision=lax.Precision.DEFAULT, preferred_element_type=jnp.float32)


def _tc1_body(js_sm, done_sm, ob_r, nx_r, ms_r,
              Wx_r, bx_r, Wm_r, bm_r, sxm_r):
  pid = pl.program_id(0)
  obs = ob_r[0]
  nxs = nx_r[:, :]
  mss = ms_r[:, :]
  for b in range(B):
    n = pid * B + b
    x_cat = jnp.concatenate(
        [obs[b:b + 1]] + [nxs[K * b + k:K * b + k + 1] for k in range(K)],
        axis=1)
    m_rows = []
    for k in range(K):
      mj = 1.0 - done_sm[js_sm[n, k]]
      m_rows.append(mss[K * b + k:K * b + k + 1, :N_H] * mj)
    m_cat = jnp.concatenate(m_rows, axis=1)

    sx = _dot(x_cat, Wx_r[b]) + bx_r[0, b:b + 1]
    sm = _dot(m_cat, Wm_r[b]) + bm_r[0, b:b + 1]
    sxm_r[:, b:b + 1, :] = (jnp.maximum(sx, 0.0) + jnp.maximum(sm, 0.0))[None]


def _tc2_body(done_sm, sxm_r, sp_r, bp_r, gh_r, st_r,
              Wi_r, bi_r, Wa_r, ba_r, lg_r, pr_r, ns_r, g_scr, c_scr, l_scr):
  pid = pl.program_id(0)
  sxm = sxm_r[0]
  sps = sp_r[:, :]
  ghs = gh_r[:, :]
  sts = st_r[0]
  s_all = sxm + jnp.maximum(sps + bp_r[0], 0.0)
  for b in range(B):
    n = pid * B + b
    msk = 1.0 - done_sm[n]
    c_scr[b:b + 1, :] = sts[b:b + 1, N_H:] * msk
    g_scr[b:b + 1, :] = (_dot(s_all[b:b + 1], Wi_r[b]) + ghs[b:b + 1] * msk
                         + bi_r[0, b:b + 1])

  gates = g_scr[:, :]
  ig = jax.nn.sigmoid(gates[:, 0:N_H])
  fg = jax.nn.sigmoid(gates[:, N_H:2 * N_H])
  gg = jnp.tanh(gates[:, 2 * N_H:3 * N_H])
  og = jax.nn.sigmoid(gates[:, 3 * N_H:4 * N_H])
  c_new = fg * c_scr[:, :] + ig * gg
  h_new = og * jnp.tanh(c_new)
  ns_r[0, :, 0:N_H] = h_new
  ns_r[0, :, N_H:2 * N_H] = c_new

  for b in range(B):
    l_scr[b:b + 1, :] = jax.lax.dot_general(
        h_new[b:b + 1], Wa_r[b], (((1,), (1,)), ((), ())),
        precision=lax.Precision.DEFAULT,
        preferred_element_type=jnp.float32) + ba_r[0, b:b + 1]
  logits = l_scr[:, :]
  mx = jnp.max(logits, axis=1, keepdims=True)
  e = jnp.exp(logits - mx)
  probs = e / jnp.sum(e, axis=1, keepdims=True)
  lg_r[0] = logits
  pr_r[0] = probs


_smem = lambda: pl.BlockSpec(memory_space=pltpu.SMEM)
_row3 = lambda d: pl.BlockSpec((1, B, d), lambda i: (i, 0, 0))
_gat2 = lambda d: pl.BlockSpec((B * K, d), lambda i: (i, 0))
_row2 = lambda d: pl.BlockSpec((B, d), lambda i: (i, 0))
_wspec = lambda a, d: pl.BlockSpec((B, a, d), lambda i: (i, 0, 0))


def _tc1_call(js, done_f, ob3, nx, ms, Wx, bx3, Wm, bm3):
  return pl.pallas_call(
      _tc1_body,
      grid=(NB,),
      in_specs=[
          _smem(), _smem(),
          _row3(N_S), _gat2(N_S), _gat2(2 * N_H),
          _wspec((K + 1) * N_S, N_FC), _row3(N_FC),
          _wspec(K * N_H, N_FC), _row3(N_FC),
      ],
      out_specs=[_row3(N_FC)],
      out_shape=[jax.ShapeDtypeStruct((NB, B, N_FC), jnp.float32)],
  )(js, done_f, ob3, nx, ms, Wx, bx3, Wm, bm3)[0]


def _tc2_call(done_f, sxm3, sp, bp3, gh, st3, Wi, bi3, Wa, ba3):
  return pl.pallas_call(
      _tc2_body,
      grid=(NB,),
      in_specs=[
          _smem(),
          _row3(N_FC), _row2(N_FC), _row3(N_FC), _row2(4 * N_H),
          _row3(2 * N_H),
          _wspec(N_FC, 4 * N_H), _row3(4 * N_H),
          _wspec(N_A, N_H), _row3(N_A),
      ],
      out_specs=[_row3(N_A), _row3(N_A), _row3(2 * N_H)],
      out_shape=[
          jax.ShapeDtypeStruct((NB, B, N_A), jnp.float32),
          jax.ShapeDtypeStruct((NB, B, N_A), jnp.float32),
          jax.ShapeDtypeStruct((NB, B, 2 * N_H), jnp.float32),
      ],
      scratch_shapes=[
          pltpu.VMEM((B, 4 * N_H), jnp.float32),
          pltpu.VMEM((B, N_H), jnp.float32),
          pltpu.VMEM((B, N_A), jnp.float32),
      ],
  )(done_f, sxm3, sp, bp3, gh, st3, Wi, bi3, Wa, ba3)


def kernel(ob, done, fp, states, js, Wx, bx, Wp, bp, Wm, bm, Wi, Wh, bi, Wa, ba):
  done_f = done.astype(jnp.float32)
  js_flat = js.reshape(N * K)
  fp_pad = jnp.pad(fp, ((0, 0), (0, 128 - N_A)))
  nx, pf, ms = _sc_gather(ob, fp_pad, states, js_flat)
  sp_raw, gh = _sc_pwh(pf, Wp, states, Wh)
  sxm3 = _tc1_call(
      js, done_f,
      ob.reshape(NB, B, N_S), nx, ms,
      Wx, bx.reshape(NB, B, N_FC), Wm, bm.reshape(NB, B, N_FC))
  lg3, pr3, ns3 = _tc2_call(
      done_f, sxm3, sp_raw, bp.reshape(NB, B, N_FC),
      gh, states.reshape(NB, B, 2 * N_H),
      Wi, bi.reshape(NB, B, 4 * N_H),
      jnp.transpose(Wa, (0, 2, 1)), ba.reshape(NB, B, N_A))
  return (lg3.reshape(N, N_A), pr3.reshape(N, N_A), ns3.reshape(N, 2 * N_H))


# speedup vs baseline: 1.1216x; 1.1216x over previous
import functools

import jax
import jax.numpy as jnp
from jax import lax
from jax.experimental import pallas as pl
from jax.experimental.pallas import tpu as pltpu
from jax.experimental.pallas import tpu_sc as plsc

N = 256
K = 4
N_S = 128
N_A = 16
N_FC = 128
N_H = 128

B = 32
NB = N // B

NW = 32
BPW = (N * K) // NW
APW = N // NW
CH = 64


def _sc_gather(ob, fp, states, js_flat):
  mesh = plsc.VectorSubcoreMesh(core_axis_name="c", subcore_axis_name="s")

  @functools.partial(
      pl.kernel, mesh=mesh,
      out_type=[
          jax.ShapeDtypeStruct((N * K, N_S), jnp.float32),
          jax.ShapeDtypeStruct((N * K, 128), jnp.float32),
          jax.ShapeDtypeStruct((N * K, 2 * N_H), jnp.float32),
      ],
      scratch_types=[
          pltpu.VMEM((BPW,), jnp.int32),
          pltpu.VMEM((BPW, N_S), jnp.float32),
          pltpu.VMEM((BPW, 128), jnp.float32),
          pltpu.VMEM((BPW, 2 * N_H), jnp.float32),
          pltpu.SemaphoreType.DMA,
          pltpu.SemaphoreType.DMA,
          pltpu.SemaphoreType.DMA,
      ],
  )
  def gather_kernel(ob_hbm, fp_hbm, st_hbm, js_hbm, nx_hbm, pf_hbm, ms_hbm,
                    idx_v, a_v, b_v, c_v, s0, s1, s2):
    wid = lax.axis_index("s") * 2 + lax.axis_index("c")
    base = wid * BPW
    pltpu.sync_copy(js_hbm.at[pl.ds(base, BPW)], idx_v)
    cp0 = pltpu.async_copy(ob_hbm.at[idx_v], a_v, s0)
    cp1 = pltpu.async_copy(fp_hbm.at[idx_v], b_v, s1)
    cp2 = pltpu.async_copy(st_hbm.at[idx_v], c_v, s2)
    cp0.wait()
    o0 = pltpu.async_copy(a_v, nx_hbm.at[pl.ds(base, BPW)], s0)
    cp1.wait()
    o1 = pltpu.async_copy(b_v, pf_hbm.at[pl.ds(base, BPW)], s1)
    cp2.wait()
    o2 = pltpu.async_copy(c_v, ms_hbm.at[pl.ds(base, BPW)], s2)
    o0.wait()
    o1.wait()
    o2.wait()

  return gather_kernel(ob, fp, states, js_flat)


def _sc_hwh(states, Wh):
  mesh = plsc.VectorSubcoreMesh(core_axis_name="c", subcore_axis_name="s")

  @functools.partial(
      pl.kernel, mesh=mesh,
      out_type=jax.ShapeDtypeStruct((N, 4 * N_H), jnp.float32),
      scratch_types=[
          pltpu.VMEM((APW, 2 * N_H), jnp.float32),
          pltpu.VMEM((CH, 4 * N_H), jnp.float32),
          pltpu.VMEM((CH, 4 * N_H), jnp.float32),
          pltpu.VMEM((APW, 4 * N_H), jnp.float32),
          pltpu.SemaphoreType.DMA,
          pltpu.SemaphoreType.DMA,
      ],
  )
  def hwh_kernel(st_hbm, wh_hbm, gh_hbm, h_all, w0, w1, out_v, s0, s1):
    wid = lax.axis_index("s") * 2 + lax.axis_index("c")
    a0 = wid * APW
    pltpu.sync_copy(st_hbm.at[pl.ds(a0, APW), :], h_all)

    bufs = (w0, w1)
    sems = (s0, s1)

    def issue(a, c):
      return pltpu.async_copy(
          wh_hbm.at[a0 + a, pl.ds(c * CH, CH), :], bufs[c], sems[c])

    def compute_chunk(a, c, w_r):
      for og in range(4):
        if c == 0:
          acc = tuple(jnp.zeros((16,), jnp.float32) for _ in range(8))
        else:
          acc = tuple(out_v[a, pl.ds(og * 128 + o * 16, 16)]
                      for o in range(8))

        def body(j, acc, a=a, c=c, og=og, w_r=w_r):
          hv = h_all[a, pl.ds(c * CH + j * 16, 16)]
          for u in range(16):
            hi = hv[u]
            acc = tuple(
                acc[o] + hi * w_r[j * 16 + u, pl.ds(og * 128 + o * 16, 16)]
                for o in range(8))
          return acc

        acc = lax.fori_loop(0, CH // 16, body, acc)
        for o in range(8):
          out_v[a, pl.ds(og * 128 + o * 16, 16)] = acc[o]

    issue(0, 0)

    def agent_body(a, carry):
      pltpu.make_async_copy(
          wh_hbm.at[a0 + a, pl.ds(0, CH), :], w0, s0).wait()
      issue(a, 1)
      compute_chunk(a, 0, w0)
      pltpu.make_async_copy(
          wh_hbm.at[a0 + a, pl.ds(CH, CH), :], w1, s1).wait()

      @pl.when(a < APW - 1)
      def _():
        issue(a + 1, 0)

      compute_chunk(a, 1, w1)
      return carry

    lax.fori_loop(0, APW, agent_body, 0)
    pltpu.sync_copy(out_v, gh_hbm.at[pl.ds(a0, APW)])

  return hwh_kernel(states, Wh)


def _dot(u, v):
  return jax.lax.dot_general(
      u, v, (((1,), (0,)), ((), ())),
      precision=lax.Precision.DEFAULT, preferred_element_type=jnp.float32)


def _tc1_body(js_sm, done_sm, ob_r, nx_r, pf_r, ms_r,
              Wx_r, bx_r, Wp_r, bp_r, Wm_r, bm_r, s_r):
  pid = pl.program_id(0)
  obs = ob_r[0]
  nxs = nx_r[:, :]
  pfs = pf_r[:, :]
  mss = ms_r[:, :]
  for b in range(B):
    n = pid * B + b
    x_cat = jnp.concatenate(
        [obs[b:b + 1]] + [nxs[K * b + k:K * b + k + 1] for k in range(K)],
        axis=1)
    p_cat = jnp.concatenate(
        [pfs[K * b + k:K * b + k + 1, :N_A] for k in range(K)],
        axis=1)
    m_rows = []
    for k in range(K):
      mj = 1.0 - done_sm[js_sm[n, k]]
      m_rows.append(mss[K * b + k:K * b + k + 1, :N_H] * mj)
    m_cat = jnp.concatenate(m_rows, axis=1)

    sx = _dot(x_cat, Wx_r[b]) + bx_r[0, b:b + 1]
    sp = _dot(p_cat, Wp_r[b]) + bp_r[0, b:b + 1]
    sm = _dot(m_cat, Wm_r[b]) + bm_r[0, b:b + 1]
    s_r[:, b:b + 1, :] = (jnp.maximum(sx, 0.0) + jnp.maximum(sp, 0.0)
                          + jnp.maximum(sm, 0.0))[None]


def _tc2_body(done_sm, s_in_r, gh_r, st_r,
              Wi_r, bi_r, Wa_r, ba_r, lg_r, pr_r, ns_r, g_scr, c_scr, l_scr):
  pid = pl.program_id(0)
  s_all = s_in_r[0]
  ghs = gh_r[:, :]
  sts = st_r[0]
  for b in range(B):
    n = pid * B + b
    msk = 1.0 - done_sm[n]
    c_scr[b:b + 1, :] = sts[b:b + 1, N_H:] * msk
    g_scr[b:b + 1, :] = (_dot(s_all[b:b + 1], Wi_r[b]) + ghs[b:b + 1] * msk
                         + bi_r[0, b:b + 1])

  gates = g_scr[:, :]
  ig = jax.nn.sigmoid(gates[:, 0:N_H])
  fg = jax.nn.sigmoid(gates[:, N_H:2 * N_H])
  gg = jnp.tanh(gates[:, 2 * N_H:3 * N_H])
  og = jax.nn.sigmoid(gates[:, 3 * N_H:4 * N_H])
  c_new = fg * c_scr[:, :] + ig * gg
  h_new = og * jnp.tanh(c_new)
  ns_r[0, :, 0:N_H] = h_new
  ns_r[0, :, N_H:2 * N_H] = c_new

  for b in range(B):
    l_scr[b:b + 1, :] = jax.lax.dot_general(
        h_new[b:b + 1], Wa_r[b], (((1,), (1,)), ((), ())),
        precision=lax.Precision.DEFAULT,
        preferred_element_type=jnp.float32) + ba_r[0, b:b + 1]
  logits = l_scr[:, :]
  mx = jnp.max(logits, axis=1, keepdims=True)
  e = jnp.exp(logits - mx)
  probs = e / jnp.sum(e, axis=1, keepdims=True)
  lg_r[0] = logits
  pr_r[0] = probs


_smem = lambda: pl.BlockSpec(memory_space=pltpu.SMEM)
_row3 = lambda d: pl.BlockSpec((1, B, d), lambda i: (i, 0, 0))
_gat2 = lambda d: pl.BlockSpec((B * K, d), lambda i: (i, 0))
_row2 = lambda d: pl.BlockSpec((B, d), lambda i: (i, 0))
_wspec = lambda a, d: pl.BlockSpec((B, a, d), lambda i: (i, 0, 0))


def _tc1_call(js, done_f, ob3, nx, pf, ms, Wx, bx3, Wp, bp3, Wm, bm3):
  return pl.pallas_call(
      _tc1_body,
      grid=(NB,),
      in_specs=[
          _smem(), _smem(),
          _row3(N_S), _gat2(N_S), _gat2(128), _gat2(2 * N_H),
          _wspec((K + 1) * N_S, N_FC), _row3(N_FC),
          _wspec(K * N_A, N_FC), _row3(N_FC),
          _wspec(K * N_H, N_FC), _row3(N_FC),
      ],
      out_specs=[_row3(N_FC)],
      out_shape=[jax.ShapeDtypeStruct((NB, B, N_FC), jnp.float32)],
  )(js, done_f, ob3, nx, pf, ms, Wx, bx3, Wp, bp3, Wm, bm3)[0]


def _tc2_call(done_f, s3, gh, st3, Wi, bi3, Wa, ba3):
  return pl.pallas_call(
      _tc2_body,
      grid=(NB,),
      in_specs=[
          _smem(),
          _row3(N_FC), _row2(4 * N_H),
          _row3(2 * N_H),
          _wspec(N_FC, 4 * N_H), _row3(4 * N_H),
          _wspec(N_A, N_H), _row3(N_A),
      ],
      out_specs=[_row3(N_A), _row3(N_A), _row3(2 * N_H)],
      out_shape=[
          jax.ShapeDtypeStruct((NB, B, N_A), jnp.float32),
          jax.ShapeDtypeStruct((NB, B, N_A), jnp.float32),
          jax.ShapeDtypeStruct((NB, B, 2 * N_H), jnp.float32),
      ],
      scratch_shapes=[
          pltpu.VMEM((B, 4 * N_H), jnp.float32),
          pltpu.VMEM((B, N_H), jnp.float32),
          pltpu.VMEM((B, N_A), jnp.float32),
      ],
  )(done_f, s3, gh, st3, Wi, bi3, Wa, ba3)


def kernel(ob, done, fp, states, js, Wx, bx, Wp, bp, Wm, bm, Wi, Wh, bi, Wa, ba):
  done_f = done.astype(jnp.float32)
  js_flat = js.reshape(N * K)
  fp_pad = jnp.pad(fp, ((0, 0), (0, 128 - N_A)))
  nx, pf, ms = _sc_gather(ob, fp_pad, states, js_flat)
  gh = _sc_hwh(states, Wh)
  s3 = _tc1_call(
      js, done_f,
      ob.reshape(NB, B, N_S), nx, pf, ms,
      Wx, bx.reshape(NB, B, N_FC), Wp, bp.reshape(NB, B, N_FC),
      Wm, bm.reshape(NB, B, N_FC))
  lg3, pr3, ns3 = _tc2_call(
      done_f, s3,
      gh, states.reshape(NB, B, 2 * N_H),
      Wi, bi.reshape(NB, B, 4 * N_H),
      jnp.transpose(Wa, (0, 2, 1)), ba.reshape(NB, B, N_A))
  return (lg3.reshape(N, N_A), pr3.reshape(N, N_A), ns3.reshape(N, 2 * N_H))
